# baseline (speedup 1.0000x reference)
import functools

import jax
import jax.numpy as jnp
from jax import lax
from jax.experimental import pallas as pl
from jax.experimental.pallas import tpu as pltpu
from jax.experimental.pallas import tpu_sc as plsc

DIM = 300
SEQ = 200
TOK = 180
BATCH = 1024
GRAN = 8
GPT = 38
SROWS = TOK * GPT
DWORDS = TOK * DIM
TOK_PAD = 192
CHUNKS = 19


@functools.lru_cache(maxsize=1)
def _make_sc_gather():
    info = plsc.get_sparse_core_info()
    nw = info.num_cores * info.num_subcores
    bpw = BATCH // nw
    mesh = plsc.VectorSubcoreMesh(core_axis_name="c", subcore_axis_name="s")

    @functools.partial(
        pl.kernel,
        mesh=mesh,
        compiler_params=pltpu.CompilerParams(
            use_tc_tiling_on_sc=False, needs_layout_passes=False),
        out_type=jax.ShapeDtypeStruct((BATCH, DWORDS), jnp.float32),
        scratch_types=[
            pltpu.VMEM((TOK_PAD,), jnp.int32),
            pltpu.VMEM((SROWS,), jnp.int32),
            pltpu.VMEM((SROWS, GRAN), jnp.float32),
            pltpu.VMEM((DWORDS + 16,), jnp.float32),
            pltpu.SemaphoreType.DMA,
            pltpu.SemaphoreType.DMA,
        ],
    )
    def gather_kernel(idx_hbm, tableg_hbm, out_hbm,
                      iv, glv, stg, pkd, gsem, ssem):
        wid = lax.axis_index("s") * info.num_cores + lax.axis_index("c")
        b0 = wid * bpw

        viota = lax.iota(jnp.int32, 16)
        viota38 = viota * GPT
        row_p0 = viota >> 3
        col_p0 = viota & 7
        row_p4 = (viota + 4) >> 3
        col_p4 = (viota + 4) & 7

        def gen_glist():
            def grp(g, c):
                base = pl.multiple_of(16 * g, 16)
                tv = iv[pl.ds(base, 16)]
                sv = (tv * 75) >> 1
                dst0 = viota38 + (GPT * 16) * g
                for k in range(GPT):
                    plsc.store_scatter(glv, [dst0 + k], sv + k)
                return c

            lax.fori_loop(0, TOK // 16, grp, 0)
            tv = iv[pl.ds(16 * (TOK // 16), 16)]
            sv = (tv * 75) >> 1
            dst0 = viota38 + (GPT * 16) * (TOK // 16)
            msk = viota < (TOK - 16 * (TOK // 16))
            for k in range(GPT):
                plsc.store_scatter(glv, [dst0 + k], sv + k, mask=msk)

        def extract():
            def token_block(g, tv, u):
                t = 16 * g + u
                phi = tv[u] & 1
                is4 = phi != 0
                srow = jnp.where(is4, row_p4, row_p0)
                scol = jnp.where(is4, col_p4, col_p0)
                sbase = GPT * t
                dbase = DIM * t
                for k in range(CHUNKS):
                    x = plsc.load_gather(stg, [srow + (sbase + 2 * k), scol])
                    plsc.store_scatter(pkd, [viota + (dbase + 16 * k)], x)

            def grp(g, c):
                base = pl.multiple_of(16 * g, 16)
                tv = iv[pl.ds(base, 16)]
                for u in range(16):
                    token_block(g, tv, u)
                return c

            lax.fori_loop(0, TOK // 16, grp, 0)
            tv = iv[pl.ds(16 * (TOK // 16), 16)]
            for u in range(TOK - 16 * (TOK // 16)):
                token_block(TOK // 16, tv, u)

        def drain_store():
            pltpu.make_async_copy(
                pkd.at[pl.ds(0, DWORDS)],
                out_hbm.at[b0, pl.ds(0, DWORDS)], ssem).wait()

        def body(j, carry):
            pltpu.sync_copy(idx_hbm.at[b0 + j], iv)
            gen_glist()
            g = pltpu.async_copy(tableg_hbm.at[glv], stg, gsem)

            @pl.when(j >= 1)
            def _():
                drain_store()

            g.wait()
            extract()
            pltpu.async_copy(pkd.at[pl.ds(0, DWORDS)],
                             out_hbm.at[b0 + j, pl.ds(0, DWORDS)], ssem)
            return carry

        lax.fori_loop(0, bpw, body, 0)
        drain_store()

    return gather_kernel


def kernel(indices, table):
    idx = jnp.pad(indices, ((0, 0), (0, TOK_PAD - TOK)))
    tg = table.reshape(-1, GRAN)
    out = _make_sc_gather()(idx, tg)
    emb = out.reshape(BATCH, TOK, DIM)
    return jnp.pad(emb, ((0, 0), (0, SEQ - TOK), (0, 0)))


# speedup vs baseline: 2.9235x; 2.9235x over previous
import functools

import jax
import jax.numpy as jnp
from jax import lax
from jax.experimental import pallas as pl
from jax.experimental.pallas import tpu as pltpu
from jax.experimental.pallas import tpu_sc as plsc

DIM = 300
SEQ = 200
TOK = 180
BATCH = 1024
TILEW = 128
SEG = 3
TPAD = 184
IPAD = 192
SROWS = SEG * TPAD
HA, HB = 96, 88


@functools.lru_cache(maxsize=1)
def _make_sc_gather():
    info = plsc.get_sparse_core_info()
    nw = info.num_cores * info.num_subcores
    bpw = BATCH // nw
    mesh = plsc.VectorSubcoreMesh(core_axis_name="c", subcore_axis_name="s")

    @functools.partial(
        pl.kernel,
        mesh=mesh,
        out_type=jax.ShapeDtypeStruct((BATCH * SROWS, TILEW), jnp.float32),
        scratch_types=[
            pltpu.VMEM((IPAD,), jnp.int32),
            pltpu.VMEM((SROWS, TILEW), jnp.float32),
            pltpu.SemaphoreType.DMA,
            pltpu.SemaphoreType.DMA,
        ],
    )
    def gather_kernel(idx_hbm, table_hbm, tail_hbm, out_hbm,
                      iv, pkd, gsem, ssem):
        wid = lax.axis_index("s") * info.num_cores + lax.axis_index("c")
        b0 = wid * bpw
        views = (
            table_hbm.at[:, pl.ds(0, TILEW)],
            table_hbm.at[:, pl.ds(TILEW, TILEW)],
            tail_hbm,
        )

        def body(j, carry):
            pltpu.sync_copy(idx_hbm.at[pl.ds(IPAD * (b0 + j), IPAD)], iv)

            @pl.when(j >= 1)
            def _():
                pltpu.make_async_copy(
                    pkd, out_hbm.at[pl.ds(SROWS * b0, SROWS)], ssem).wait()

            descs = []
            for ct in range(SEG):
                for off, n in ((0, HA), (HA, HB)):
                    descs.append(pltpu.async_copy(
                        views[ct].at[iv.at[pl.ds(off, n)]],
                        pkd.at[pl.ds(TPAD * ct + off, n)], gsem))
            for d in descs:
                d.wait()
            pltpu.async_copy(
                pkd, out_hbm.at[pl.ds(SROWS * (b0 + j), SROWS)], ssem)
            return carry

        lax.fori_loop(0, bpw, body, 0)
        pltpu.make_async_copy(
            pkd, out_hbm.at[pl.ds(SROWS * b0, SROWS)], ssem).wait()

    return gather_kernel


def kernel(indices, table):
    idx = jnp.pad(indices, ((0, 0), (0, IPAD - TOK))).reshape(-1)
    tail = table[:, DIM - TILEW:]
    out = _make_sc_gather()(idx, table, tail)
    o = out.reshape(BATCH, SEG, TPAD, TILEW)
    emb = jnp.concatenate(
        [o[:, 0, :TOK], o[:, 1, :TOK],
         o[:, 2, :TOK, TILEW - (DIM - 2 * TILEW):]], axis=-1)
    return jnp.pad(emb, ((0, 0), (0, SEQ - TOK), (0, 0)))
